# Initial kernel scaffold; baseline (speedup 1.0000x reference)
import jax
import jax.numpy as jnp
from jax.experimental import pallas as pl


def kernel(x, pos_table):
    raise NotImplementedError("write your pallas kernel here")


# speedup vs baseline: 5.2030x; 5.2030x over previous
import jax
import jax.numpy as jnp
from jax.experimental import pallas as pl


def _broadcast_body(tab_ref, out_ref):
    t = tab_ref[...]
    for b in range(out_ref.shape[0]):
        out_ref[b] = t


def kernel(x, pos_table):
    b, n = x.shape[0], x.shape[1]
    d = pos_table.shape[1]
    bn = 512
    return pl.pallas_call(
        _broadcast_body,
        grid=(n // bn,),
        in_specs=[pl.BlockSpec((bn, d), lambda i: (i, 0))],
        out_specs=pl.BlockSpec((b, bn, d), lambda i: (0, i, 0)),
        out_shape=jax.ShapeDtypeStruct((b, n, d), pos_table.dtype),
    )(pos_table)


# speedup vs baseline: 5.2055x; 1.0005x over previous
import jax
import jax.numpy as jnp
from jax.experimental import pallas as pl
from jax.experimental.pallas import tpu as pltpu


def _broadcast_body(tab_ref, out_ref):
    t = tab_ref[...]
    for b in range(out_ref.shape[0]):
        out_ref[b] = t


def kernel(x, pos_table):
    b, n = x.shape[0], x.shape[1]
    d = pos_table.shape[1]
    bn = 512
    return pl.pallas_call(
        _broadcast_body,
        grid=(n // bn,),
        in_specs=[pl.BlockSpec((bn, d), lambda i: (i, 0))],
        out_specs=pl.BlockSpec((b, bn, d), lambda i: (0, i, 0)),
        out_shape=jax.ShapeDtypeStruct((b, n, d), pos_table.dtype),
        compiler_params=pltpu.CompilerParams(
            dimension_semantics=("parallel",)
        ),
    )(pos_table)


# speedup vs baseline: 5.3361x; 1.0251x over previous
import jax
import jax.numpy as jnp
from jax.experimental import pallas as pl
from jax.experimental.pallas import tpu as pltpu


def _broadcast_body(tab_ref, out_ref):
    t = tab_ref[...]
    for b in range(out_ref.shape[0]):
        out_ref[b] = t


def kernel(x, pos_table):
    b, n = x.shape[0], x.shape[1]
    d = pos_table.shape[1]
    bn = 1024
    return pl.pallas_call(
        _broadcast_body,
        grid=(n // bn,),
        in_specs=[pl.BlockSpec((bn, d), lambda i: (i, 0))],
        out_specs=pl.BlockSpec((b, bn, d), lambda i: (0, i, 0)),
        out_shape=jax.ShapeDtypeStruct((b, n, d), pos_table.dtype),
        compiler_params=pltpu.CompilerParams(
            dimension_semantics=("parallel",)
        ),
    )(pos_table)
